# baseline (speedup 1.0000x reference)
import functools

import jax
import jax.numpy as jnp
from jax import lax
from jax.experimental import pallas as pl
from jax.experimental.pallas import tpu as pltpu
from jax.experimental.pallas import tpu_sc as plsc

_VOCAB = 100000
_EMBED = 32
_BATCH = 1024
_CTX = 20

_NC = 2
_NS = 16
_NW = _NC * _NS
_ROWS_PER_W = _BATCH // _NW
_IDX_PER_W = _ROWS_PER_W * _CTX
_CHUNK = 128
_NCHUNK = _IDX_PER_W // _CHUNK

_HALF = 16


def _sc_avg_body(idx_hbm, table_hbm, avg_hbm, idx_v, rows_v, avg_v, sem):
    wid = lax.axis_index("s") * _NC + lax.axis_index("c")

    pltpu.sync_copy(idx_hbm.at[pl.ds(wid * _IDX_PER_W, _IDX_PER_W)], idx_v)

    copies = []
    for j in range(_NCHUNK):
        copies.append(
            pltpu.async_copy(
                table_hbm.at[idx_v.at[pl.ds(j * _CHUNK, _CHUNK)]],
                rows_v.at[pl.ds(j * _CHUNK, _CHUNK)],
                sem,
            )
        )
    for c in copies:
        c.wait()

    inv = jnp.float32(1.0 / _CTX)

    def body(i, carry):
        acc0 = jnp.zeros((_HALF,), jnp.float32)
        acc1 = jnp.zeros((_HALF,), jnp.float32)
        for c in range(_CTX):
            r = i * _CTX + c
            acc0 = acc0 + rows_v[r, pl.ds(0, _HALF)]
            acc1 = acc1 + rows_v[r, pl.ds(_HALF, _HALF)]
        avg_v[i, pl.ds(0, _HALF)] = acc0 * inv
        avg_v[i, pl.ds(_HALF, _HALF)] = acc1 * inv
        return carry

    lax.fori_loop(0, _ROWS_PER_W, body, 0)

    pltpu.sync_copy(avg_v, avg_hbm.at[pl.ds(wid * _ROWS_PER_W, _ROWS_PER_W)])


@functools.partial(
    pl.kernel,
    mesh=plsc.VectorSubcoreMesh(core_axis_name="c", subcore_axis_name="s"),
    out_type=jax.ShapeDtypeStruct((_BATCH, _EMBED), jnp.float32),
    scratch_types=[
        pltpu.VMEM((_IDX_PER_W,), jnp.int32),
        pltpu.VMEM((_IDX_PER_W, _EMBED), jnp.float32),
        pltpu.VMEM((_ROWS_PER_W, _EMBED), jnp.float32),
        pltpu.SemaphoreType.DMA,
    ],
    compiler_params=pltpu.CompilerParams(use_tc_tiling_on_sc=False),
)
def _sc_avg(idx_hbm, table_hbm, avg_hbm, idx_v, rows_v, avg_v, sem):
    _sc_avg_body(idx_hbm, table_hbm, avg_hbm, idx_v, rows_v, avg_v, sem)


_VB = 2048


def _matmul_body(avg_ref, w_ref, b_ref, out_ref):
    out_ref[...] = (
        lax.dot_general(
            avg_ref[...],
            w_ref[...],
            (((1,), (1,)), ((), ())),
            preferred_element_type=jnp.float32,
        )
        + b_ref[...]
    )


def _tc_out(avg, W, b2):
    return pl.pallas_call(
        _matmul_body,
        grid=(pl.cdiv(_VOCAB, _VB),),
        in_specs=[
            pl.BlockSpec((_BATCH, _EMBED), lambda i: (0, 0)),
            pl.BlockSpec((_VB, _EMBED), lambda i: (i, 0)),
            pl.BlockSpec((1, _VB), lambda i: (0, i)),
        ],
        out_specs=pl.BlockSpec((_BATCH, _VB), lambda i: (0, i)),
        out_shape=jax.ShapeDtypeStruct((_BATCH, _VOCAB), jnp.float32),
    )(avg, W, b2)


def kernel(x, emb_table, W, b):
    idx = x.astype(jnp.int32).reshape(_BATCH * _CTX)
    avg = _sc_avg(idx, emb_table)
    return _tc_out(avg, W, b.reshape(1, _VOCAB))


# speedup vs baseline: 3.0270x; 3.0270x over previous
import functools

import jax
import jax.numpy as jnp
from jax import lax
from jax.experimental import pallas as pl
from jax.experimental.pallas import tpu as pltpu
from jax.experimental.pallas import tpu_sc as plsc

_VOCAB = 100000
_EMBED = 32
_BATCH = 1024
_CTX = 20

_NC = 2
_NS = 16
_NW = _NC * _NS
_CHUNK = 128
_KCHUNK = _BATCH // _CHUNK

_HALF = 16


def _sc_avg_body(idx_hbm, table_hbm, avgt_hbm, idx_v, val_v, avg_v, sem):
    wid = lax.axis_index("s") * _NC + lax.axis_index("c")

    pltpu.sync_copy(idx_hbm, idx_v)

    row = table_hbm.at[wid]

    def fire(c, carry):
        for k in range(_KCHUNK):
            pltpu.async_copy(
                row.at[idx_v.at[c, pl.ds(k * _CHUNK, _CHUNK)]],
                val_v.at[c, pl.ds(k * _CHUNK, _CHUNK)],
                sem,
            )
        return carry

    lax.fori_loop(0, _CTX, fire, 0)

    pltpu.make_async_copy(
        table_hbm.at[pl.ds(0, _CTX), pl.ds(0, _BATCH)], val_v, sem
    ).wait()

    inv = jnp.float32(1.0 / _CTX)

    def body(g, carry):
        o = g * _HALF
        acc = jnp.zeros((_HALF,), jnp.float32)
        for c in range(_CTX):
            acc = acc + val_v[c, pl.ds(o, _HALF)]
        avg_v[pl.ds(o, _HALF)] = acc * inv
        return carry

    lax.fori_loop(0, _BATCH // _HALF, body, 0)

    pltpu.sync_copy(avg_v, avgt_hbm.at[wid])


@functools.partial(
    pl.kernel,
    mesh=plsc.VectorSubcoreMesh(core_axis_name="c", subcore_axis_name="s"),
    out_type=jax.ShapeDtypeStruct((_EMBED, _BATCH), jnp.float32),
    scratch_types=[
        pltpu.VMEM((_CTX, _BATCH), jnp.int32),
        pltpu.VMEM((_CTX, _BATCH), jnp.float32),
        pltpu.VMEM((_BATCH,), jnp.float32),
        pltpu.SemaphoreType.DMA,
    ],
    compiler_params=pltpu.CompilerParams(use_tc_tiling_on_sc=False),
)
def _sc_avg(idx_hbm, table_hbm, avgt_hbm, idx_v, val_v, avg_v, sem):
    _sc_avg_body(idx_hbm, table_hbm, avgt_hbm, idx_v, val_v, avg_v, sem)


_VB = 2560


def _matmul_body(avgt_ref, wt_ref, b_ref, out_ref):
    acc = lax.dot_general(
        wt_ref[...],
        avgt_ref[...],
        (((0,), (0,)), ((), ())),
        preferred_element_type=jnp.float32,
    )
    out_ref[...] = acc + b_ref[...].T


def _tc_out(avgt, Wt, b2):
    return pl.pallas_call(
        _matmul_body,
        grid=(pl.cdiv(_VOCAB, _VB),),
        in_specs=[
            pl.BlockSpec((_EMBED, _BATCH), lambda i: (0, 0)),
            pl.BlockSpec((_EMBED, _VB), lambda i: (0, i)),
            pl.BlockSpec((1, _VB), lambda i: (0, i)),
        ],
        out_specs=pl.BlockSpec((_VB, _BATCH), lambda i: (i, 0)),
        out_shape=jax.ShapeDtypeStruct((_VOCAB, _BATCH), jnp.float32),
    )(avgt, Wt, b2)


def kernel(x, emb_table, W, b):
    idxT = x.astype(jnp.int32).T
    tableT = emb_table.T
    avgt = _sc_avg(idxT, tableT)
    outT = _tc_out(avgt, W.T, b.reshape(1, _VOCAB))
    return outT.T
